# baseline (speedup 1.0000x reference)
import functools

import jax
import jax.numpy as jnp
from jax import lax
from jax.experimental import pallas as pl
from jax.experimental.pallas import tpu as pltpu
from jax.experimental.pallas import tpu_sc as plsc

_STRIDE = 8.0
_MIN_SIZE = 16.0
_MAX_SIZE = 256.0

_B, _H, _W = 16, 64, 64
_HW = _H * _W
_PW = _W + 2
_PAD_TILE = ((_PW * (_H + 2) + 15) // 16) * 16
_CAP = 1040
_K = 1000
_KPAD = 1008
_RT = 5040
_CHUNKS = _HW // 16


def _sigmoid(x):
    return 1.0 / (1.0 + jnp.exp(-x))


def _body(route_hbm, scale_hbm, unc_hbm, h_hbm, w_hbm, rois_hbm, sco_hbm,
          route_v, scale_v, unc_v, spad_v, ck_v, ci_v, rois_t, sco_t, h_v, w_v):
    wid = lax.axis_index("c") * 16 + lax.axis_index("s")
    b = wid

    @pl.when(wid < _B)
    def _work():
        pltpu.sync_copy(route_hbm.at[b], route_v)
        pltpu.sync_copy(scale_hbm.at[b], scale_v)
        pltpu.sync_copy(unc_hbm.at[b], unc_v)
        pltpu.sync_copy(h_hbm, h_v)
        pltpu.sync_copy(w_hbm, w_v)

        iota16 = lax.iota(jnp.int32, 16)
        zf16 = jnp.zeros((16,), jnp.float32)

        def _fill(i, _):
            spad_v[pl.ds(i * 16, 16)] = zf16 - 1.0
            return 0
        lax.fori_loop(0, _PAD_TILE // 16, _fill, 0)

        def _score(t, _):
            r = route_v[pl.ds(t * 16, 16)]
            u = unc_v[pl.ds(t * 16, 16)]
            sr = _sigmoid(r)
            sc = sr * sr * (1.0 - 0.35 * _sigmoid(u))
            y = t >> 2
            q = t & 3
            spad_v[pl.ds((y + 1) * _PW + 1 + q * 16, 16)] = sc
            return 0
        lax.fori_loop(0, _CHUNKS, _score, 0)

        def _fillck(i, _):
            ck_v[pl.ds(i * 16, 16)] = zf16 - 1.0
            ci_v[pl.ds(i * 16, 16)] = iota16 * 0
            return 0
        lax.fori_loop(0, _CAP // 16, _fillck, 0)

        def _compact(t, off):
            y = t >> 2
            q = t & 3
            o = (y + 1) * _PW + 1 + q * 16
            sc = spad_v[pl.ds(o, 16)]
            m = jnp.maximum(spad_v[pl.ds(o - 1, 16)], spad_v[pl.ds(o + 1, 16)])
            m = jnp.maximum(m, jnp.maximum(spad_v[pl.ds(o - _PW - 1, 16)],
                                           spad_v[pl.ds(o - _PW, 16)]))
            m = jnp.maximum(m, jnp.maximum(spad_v[pl.ds(o - _PW + 1, 16)],
                                           spad_v[pl.ds(o + _PW - 1, 16)]))
            m = jnp.maximum(m, jnp.maximum(spad_v[pl.ds(o + _PW, 16)],
                                           spad_v[pl.ds(o + _PW + 1, 16)]))
            mask = sc >= m
            pos = off + plsc.cumsum(mask.astype(jnp.int32)) - 1
            mask = mask & (pos < _CAP)
            cnt = plsc.all_reduce_population_count(mask)
            idx = t * 16 + iota16
            plsc.store_scatter(ck_v, [pos], sc, mask=mask)
            plsc.store_scatter(ci_v, [pos], idx, mask=mask)
            return off + cnt
        offv = lax.fori_loop(0, _CHUNKS, _compact, jnp.zeros((16,), jnp.int32))
        m_count = jnp.max(offv)
        mc = (m_count + 15) >> 4

        def _zero(i, _):
            rois_t[pl.ds(i * 16, 16)] = zf16
            return 0
        lax.fori_loop(0, _RT // 16, _zero, 0)

        def _zero2(i, _):
            sco_t[pl.ds(i * 16, 16)] = zf16
            return 0
        lax.fori_loop(0, _KPAD // 16, _zero2, 0)

        img_h = h_v[...]
        img_w = w_v[...]
        bf = zf16 + lax.convert_element_type(b, jnp.float32)

        rots = [(iota16 + r) & 15 for r in range(16)]

        def _rank(ci, _):
            vi = ck_v[pl.ds(ci * 16, 16)]
            ii = ci_v[pl.ds(ci * 16, 16)]
            pi = ci * 16 + iota16

            def _inner(cj, cnts):
                base_j = cj * 16
                ck_s = ck_v.at[pl.ds(base_j, 16)]
                pib = pi - base_j
                acc = list(cnts)
                for r in range(16):
                    kj = plsc.load_gather(ck_s, [rots[r]])
                    beats = (kj > vi) | ((kj == vi) & (rots[r] < pib))
                    acc[r & 3] = acc[r & 3] + beats.astype(jnp.int32)
                return tuple(acc)
            z4 = (jnp.zeros((16,), jnp.int32),) * 4
            c0, c1, c2, c3 = lax.fori_loop(0, mc, _inner, z4)
            rank = (c0 + c1) + (c2 + c3)

            ok = (vi > 0.0) & (rank < _K)
            rows = jnp.minimum(rank, _K)
            ii_safe = ii & (_HW - 1)
            xs = (ii_safe & (_W - 1)).astype(jnp.float32)
            ys = (ii_safe >> 6).astype(jnp.float32)
            cx = (xs + 0.5) * _STRIDE
            cy = (ys + 0.5) * _STRIDE
            sl = plsc.load_gather(scale_v, [ii_safe])
            ul = plsc.load_gather(unc_v, [ii_safe])
            side = (_MIN_SIZE + _sigmoid(sl) * (_MAX_SIZE - _MIN_SIZE)) \
                * (1.0 + 0.25 * _sigmoid(ul))
            half = side * 0.5
            x1 = jnp.maximum(jnp.minimum(cx - half, img_w - 1.0), 0.0)
            y1 = jnp.maximum(jnp.minimum(cy - half, img_h - 1.0), 0.0)
            x2 = jnp.maximum(jnp.minimum(cx + half, img_w), 1.0)
            y2 = jnp.maximum(jnp.minimum(cy + half, img_h), 1.0)
            r5 = rows * 5
            plsc.store_scatter(rois_t, [r5], bf, mask=ok)
            plsc.store_scatter(rois_t, [r5 + 1], x1, mask=ok)
            plsc.store_scatter(rois_t, [r5 + 2], y1, mask=ok)
            plsc.store_scatter(rois_t, [r5 + 3], x2, mask=ok)
            plsc.store_scatter(rois_t, [r5 + 4], y2, mask=ok)
            plsc.store_scatter(sco_t, [rows], vi, mask=ok)
            return 0
        lax.fori_loop(0, mc, _rank, 0)

        pltpu.sync_copy(rois_t.at[pl.ds(0, _K * 5)],
                        rois_hbm.at[pl.ds(b * _K * 5, _K * 5)])
        pltpu.sync_copy(sco_t.at[pl.ds(0, _K)], sco_hbm.at[pl.ds(b * _K, _K)])


_sc_call = functools.partial(
    pl.kernel,
    out_type=(
        jax.ShapeDtypeStruct((_B * _K * 5,), jnp.float32),
        jax.ShapeDtypeStruct((_B * _K,), jnp.float32),
    ),
    mesh=plsc.VectorSubcoreMesh(core_axis_name="c", subcore_axis_name="s"),
    compiler_params=pltpu.CompilerParams(needs_layout_passes=False),
    scratch_types=[
        pltpu.VMEM((_HW,), jnp.float32),
        pltpu.VMEM((_HW,), jnp.float32),
        pltpu.VMEM((_HW,), jnp.float32),
        pltpu.VMEM((_PAD_TILE,), jnp.float32),
        pltpu.VMEM((_CAP,), jnp.float32),
        pltpu.VMEM((_CAP,), jnp.int32),
        pltpu.VMEM((_RT,), jnp.float32),
        pltpu.VMEM((_KPAD,), jnp.float32),
        pltpu.VMEM((16,), jnp.float32),
        pltpu.VMEM((16,), jnp.float32),
    ],
)(_body)


def kernel(route_logits, scale_logits, uncertainty_logits, image_h, image_w):
    B = route_logits.shape[0]
    r2 = route_logits.reshape(B, -1)
    s2 = scale_logits.reshape(B, -1)
    u2 = uncertainty_logits.reshape(B, -1)
    hvec = jnp.full((16,), jnp.asarray(image_h, jnp.float32))
    wvec = jnp.full((16,), jnp.asarray(image_w, jnp.float32))
    rois_flat, scores = _sc_call(r2, s2, u2, hvec, wvec)
    return rois_flat.reshape(B * _K, 5), scores


# speedup vs baseline: 1.2846x; 1.2846x over previous
import functools

import jax
import jax.numpy as jnp
from jax import lax
from jax.experimental import pallas as pl
from jax.experimental.pallas import tpu as pltpu
from jax.experimental.pallas import tpu_sc as plsc

_STRIDE = 8.0
_MIN_SIZE = 16.0
_MAX_SIZE = 256.0

_B, _H, _W = 16, 64, 64
_HW = _H * _W
_PW = _W + 2
_PAD_TILE = ((_PW * (_H + 2) + 15) // 16) * 16
_CAP = 1040
_CKPAD = 16
_CKSZ = 1088
_K = 1000
_KPAD = 1008
_RT = 5040
_CHUNKS = _HW // 16


def _sigmoid(x):
    return 1.0 / (1.0 + jnp.exp(-x))


def _body(route_hbm, scale_hbm, unc_hbm, h_hbm, w_hbm, rois_hbm, sco_hbm,
          route_v, scale_v, unc_v, spad_v, ck_v, ci_v, rois_t, sco_t, h_v, w_v):
    wid = lax.axis_index("c") * 16 + lax.axis_index("s")
    b = wid

    @pl.when(wid < _B)
    def _work():
        pltpu.sync_copy(route_hbm.at[b], route_v)
        pltpu.sync_copy(scale_hbm.at[b], scale_v)
        pltpu.sync_copy(unc_hbm.at[b], unc_v)
        pltpu.sync_copy(h_hbm, h_v)
        pltpu.sync_copy(w_hbm, w_v)

        iota16 = lax.iota(jnp.int32, 16)
        zf16 = jnp.zeros((16,), jnp.float32)

        def _fill(i, _):
            spad_v[pl.ds(i * 16, 16)] = zf16 - 1.0
            return 0
        lax.fori_loop(0, _PAD_TILE // 16, _fill, 0)

        def _score(t, _):
            r = route_v[pl.ds(t * 16, 16)]
            u = unc_v[pl.ds(t * 16, 16)]
            sr = _sigmoid(r)
            sc = sr * sr * (1.0 - 0.35 * _sigmoid(u))
            y = t >> 2
            q = t & 3
            spad_v[pl.ds((y + 1) * _PW + 1 + q * 16, 16)] = sc
            return 0
        lax.fori_loop(0, _CHUNKS, _score, 0)

        def _fillck(i, _):
            ck_v[pl.ds(i * 16, 16)] = zf16 - 1.0
            return 0
        lax.fori_loop(0, _CKSZ // 16, _fillck, 0)

        def _compact(t, off):
            y = t >> 2
            q = t & 3
            o = (y + 1) * _PW + 1 + q * 16
            sc = spad_v[pl.ds(o, 16)]
            m = jnp.maximum(spad_v[pl.ds(o - 1, 16)], spad_v[pl.ds(o + 1, 16)])
            m = jnp.maximum(m, jnp.maximum(spad_v[pl.ds(o - _PW - 1, 16)],
                                           spad_v[pl.ds(o - _PW, 16)]))
            m = jnp.maximum(m, jnp.maximum(spad_v[pl.ds(o - _PW + 1, 16)],
                                           spad_v[pl.ds(o + _PW - 1, 16)]))
            m = jnp.maximum(m, jnp.maximum(spad_v[pl.ds(o + _PW, 16)],
                                           spad_v[pl.ds(o + _PW + 1, 16)]))
            mask = sc >= m
            pos = off + plsc.cumsum(mask.astype(jnp.int32)) + (_CKPAD - 1)
            mask = mask & (pos < _CKPAD + _CAP)
            cnt = plsc.all_reduce_population_count(mask)
            idx = t * 16 + iota16
            plsc.store_scatter(ck_v, [pos], sc, mask=mask)
            plsc.store_scatter(ci_v, [pos], idx, mask=mask)
            return off + cnt
        offv = lax.fori_loop(0, _CHUNKS, _compact, jnp.zeros((16,), jnp.int32))
        m_count = jnp.max(offv)
        mc = (m_count + 15) >> 4

        def _zero(i, _):
            rois_t[pl.ds(i * 16, 16)] = zf16
            return 0
        lax.fori_loop(0, _RT // 16, _zero, 0)

        def _zero2(i, _):
            sco_t[pl.ds(i * 16, 16)] = zf16
            return 0
        lax.fori_loop(0, _KPAD // 16, _zero2, 0)

        img_h = h_v[...]
        img_w = w_v[...]
        bf = zf16 + lax.convert_element_type(b, jnp.float32)

        def _rank(ci, _):
            di = _CKPAD + ci * 16
            vi = ck_v[pl.ds(di, 16)]
            ii = ci_v[pl.ds(di, 16)]

            def _ge(b8, cnts):
                u = b8 * 8
                acc = list(cnts)
                for k in range(8):
                    kj = ck_v[pl.ds(u + k, 16)]
                    acc[k & 3] = acc[k & 3] + (kj >= vi).astype(jnp.int32)
                return tuple(acc)

            def _gt(b8, cnts):
                u = di + b8 * 8
                acc = list(cnts)
                for k in range(8):
                    kj = ck_v[pl.ds(u + k, 16)]
                    acc[k & 3] = acc[k & 3] + (kj > vi).astype(jnp.int32)
                return tuple(acc)

            z4 = (jnp.zeros((16,), jnp.int32),) * 4
            cnts = lax.fori_loop(0, di >> 3, _ge, z4)
            c0, c1, c2, c3 = lax.fori_loop(0, (mc - ci) * 2, _gt, cnts)
            rank = (c0 + c1) + (c2 + c3)

            ok = (vi > 0.0) & (rank < _K)
            rows = jnp.minimum(rank, _K)
            ii_safe = ii & (_HW - 1)
            xs = (ii_safe & (_W - 1)).astype(jnp.float32)
            ys = (ii_safe >> 6).astype(jnp.float32)
            cx = (xs + 0.5) * _STRIDE
            cy = (ys + 0.5) * _STRIDE
            sl = plsc.load_gather(scale_v, [ii_safe])
            ul = plsc.load_gather(unc_v, [ii_safe])
            side = (_MIN_SIZE + _sigmoid(sl) * (_MAX_SIZE - _MIN_SIZE)) \
                * (1.0 + 0.25 * _sigmoid(ul))
            half = side * 0.5
            x1 = jnp.maximum(jnp.minimum(cx - half, img_w - 1.0), 0.0)
            y1 = jnp.maximum(jnp.minimum(cy - half, img_h - 1.0), 0.0)
            x2 = jnp.maximum(jnp.minimum(cx + half, img_w), 1.0)
            y2 = jnp.maximum(jnp.minimum(cy + half, img_h), 1.0)
            r5 = rows * 5
            plsc.store_scatter(rois_t, [r5], bf, mask=ok)
            plsc.store_scatter(rois_t, [r5 + 1], x1, mask=ok)
            plsc.store_scatter(rois_t, [r5 + 2], y1, mask=ok)
            plsc.store_scatter(rois_t, [r5 + 3], x2, mask=ok)
            plsc.store_scatter(rois_t, [r5 + 4], y2, mask=ok)
            plsc.store_scatter(sco_t, [rows], vi, mask=ok)
            return 0
        lax.fori_loop(0, mc, _rank, 0)

        pltpu.sync_copy(rois_t.at[pl.ds(0, _K * 5)],
                        rois_hbm.at[pl.ds(b * _K * 5, _K * 5)])
        pltpu.sync_copy(sco_t.at[pl.ds(0, _K)], sco_hbm.at[pl.ds(b * _K, _K)])


_sc_call = functools.partial(
    pl.kernel,
    out_type=(
        jax.ShapeDtypeStruct((_B * _K * 5,), jnp.float32),
        jax.ShapeDtypeStruct((_B * _K,), jnp.float32),
    ),
    mesh=plsc.VectorSubcoreMesh(core_axis_name="c", subcore_axis_name="s"),
    compiler_params=pltpu.CompilerParams(needs_layout_passes=False),
    scratch_types=[
        pltpu.VMEM((_HW,), jnp.float32),
        pltpu.VMEM((_HW,), jnp.float32),
        pltpu.VMEM((_HW,), jnp.float32),
        pltpu.VMEM((_PAD_TILE,), jnp.float32),
        pltpu.VMEM((_CKSZ,), jnp.float32),
        pltpu.VMEM((_CKPAD + _CAP,), jnp.int32),
        pltpu.VMEM((_RT,), jnp.float32),
        pltpu.VMEM((_KPAD,), jnp.float32),
        pltpu.VMEM((16,), jnp.float32),
        pltpu.VMEM((16,), jnp.float32),
    ],
)(_body)


def kernel(route_logits, scale_logits, uncertainty_logits, image_h, image_w):
    B = route_logits.shape[0]
    r2 = route_logits.reshape(B, -1)
    s2 = scale_logits.reshape(B, -1)
    u2 = uncertainty_logits.reshape(B, -1)
    hvec = jnp.full((16,), jnp.asarray(image_h, jnp.float32))
    wvec = jnp.full((16,), jnp.asarray(image_w, jnp.float32))
    rois_flat, scores = _sc_call(r2, s2, u2, hvec, wvec)
    return rois_flat.reshape(B * _K, 5), scores


# speedup vs baseline: 1.3737x; 1.0693x over previous
import functools

import jax
import jax.numpy as jnp
from jax import lax
from jax.experimental import pallas as pl
from jax.experimental.pallas import tpu as pltpu
from jax.experimental.pallas import tpu_sc as plsc

_STRIDE = 8.0
_MIN_SIZE = 16.0
_MAX_SIZE = 256.0

_B, _H, _W = 16, 64, 64
_HW = _H * _W
_PW = _W + 2
_PAD_TILE = ((_PW * (_H + 2) + 15) // 16) * 16
_CAP = 1040
_CKPAD = 16
_CKSZ = 1088
_K = 1000
_KPAD = 1008
_RT = 5040
_CHUNKS = _HW // 16


def _sigmoid(x):
    return 1.0 / (1.0 + jnp.exp(-x))


def _body(route_hbm, scale_hbm, unc_hbm, h_hbm, w_hbm, rois_hbm, sco_hbm,
          route_v, scale_v, unc_v, spad_v, ck_v, ci_v, rois_t, sco_t, h_v, w_v):
    wid = lax.axis_index("c") * 16 + lax.axis_index("s")
    b = wid

    @pl.when(wid < _B)
    def _work():
        pltpu.sync_copy(route_hbm.at[b], route_v)
        pltpu.sync_copy(scale_hbm.at[b], scale_v)
        pltpu.sync_copy(unc_hbm.at[b], unc_v)
        pltpu.sync_copy(h_hbm, h_v)
        pltpu.sync_copy(w_hbm, w_v)

        iota16 = lax.iota(jnp.int32, 16)
        zf16 = jnp.zeros((16,), jnp.float32)

        def _fill(i, _):
            spad_v[pl.ds(i * 16, 16)] = zf16 - 1.0
            return 0
        lax.fori_loop(0, _PAD_TILE // 16, _fill, 0)

        def _score(t, _):
            r = route_v[pl.ds(t * 16, 16)]
            u = unc_v[pl.ds(t * 16, 16)]
            sr = _sigmoid(r)
            sc = sr * sr * (1.0 - 0.35 * _sigmoid(u))
            y = t >> 2
            q = t & 3
            spad_v[pl.ds((y + 1) * _PW + 1 + q * 16, 16)] = sc
            return 0
        lax.fori_loop(0, _CHUNKS, _score, 0)

        def _fillck(i, _):
            ck_v[pl.ds(i * 16, 16)] = zf16 - 1.0
            return 0
        lax.fori_loop(0, _CKSZ // 16, _fillck, 0)

        def _compact(t, off):
            y = t >> 2
            q = t & 3
            o = (y + 1) * _PW + 1 + q * 16
            sc = spad_v[pl.ds(o, 16)]
            m = jnp.maximum(spad_v[pl.ds(o - 1, 16)], spad_v[pl.ds(o + 1, 16)])
            m = jnp.maximum(m, jnp.maximum(spad_v[pl.ds(o - _PW - 1, 16)],
                                           spad_v[pl.ds(o - _PW, 16)]))
            m = jnp.maximum(m, jnp.maximum(spad_v[pl.ds(o - _PW + 1, 16)],
                                           spad_v[pl.ds(o + _PW - 1, 16)]))
            m = jnp.maximum(m, jnp.maximum(spad_v[pl.ds(o + _PW, 16)],
                                           spad_v[pl.ds(o + _PW + 1, 16)]))
            mask = sc >= m
            pos = off + plsc.cumsum(mask.astype(jnp.int32)) + (_CKPAD - 1)
            mask = mask & (pos < _CKPAD + _CAP)
            cnt = plsc.all_reduce_population_count(mask)
            idx = t * 16 + iota16
            plsc.store_scatter(ck_v, [pos], sc, mask=mask)
            plsc.store_scatter(ci_v, [pos], idx, mask=mask)
            return off + cnt
        offv = lax.fori_loop(0, _CHUNKS, _compact, jnp.zeros((16,), jnp.int32))
        m_count = jnp.max(offv)
        mc = (m_count + 15) >> 4

        def _zero(i, _):
            rows = i * 16 + iota16
            for c in range(5):
                plsc.store_scatter(rois_t, [rows, iota16 * 0 + c], zf16)
            return 0
        lax.fori_loop(0, _KPAD // 16, _zero, 0)

        def _zero2(i, _):
            sco_t[pl.ds(i * 16, 16)] = zf16
            return 0
        lax.fori_loop(0, _KPAD // 16, _zero2, 0)

        img_h = h_v[...]
        img_w = w_v[...]
        bf = zf16 + lax.convert_element_type(b, jnp.float32)

        def _rank(ci, _):
            di = _CKPAD + ci * 16
            vi = ck_v[pl.ds(di, 16)]
            ii = ci_v[pl.ds(di, 16)]

            def _ge(b8, cnts):
                u = b8 * 8
                acc = list(cnts)
                for k in range(8):
                    kj = ck_v[pl.ds(u + k, 16)]
                    acc[k & 3] = acc[k & 3] + (kj >= vi).astype(jnp.int32)
                return tuple(acc)

            def _gt(b8, cnts):
                u = di + b8 * 8
                acc = list(cnts)
                for k in range(8):
                    kj = ck_v[pl.ds(u + k, 16)]
                    acc[k & 3] = acc[k & 3] + (kj > vi).astype(jnp.int32)
                return tuple(acc)

            z4 = (jnp.zeros((16,), jnp.int32),) * 4
            cnts = lax.fori_loop(0, di >> 3, _ge, z4)
            c0, c1, c2, c3 = lax.fori_loop(0, (mc - ci) * 2, _gt, cnts)
            rank = (c0 + c1) + (c2 + c3)

            ok = (vi > 0.0) & (rank < _K)
            rows = jnp.minimum(rank, _K)
            ii_safe = ii & (_HW - 1)
            xs = (ii_safe & (_W - 1)).astype(jnp.float32)
            ys = (ii_safe >> 6).astype(jnp.float32)
            cx = (xs + 0.5) * _STRIDE
            cy = (ys + 0.5) * _STRIDE
            sl = plsc.load_gather(scale_v, [ii_safe])
            ul = plsc.load_gather(unc_v, [ii_safe])
            side = (_MIN_SIZE + _sigmoid(sl) * (_MAX_SIZE - _MIN_SIZE)) \
                * (1.0 + 0.25 * _sigmoid(ul))
            half = side * 0.5
            x1 = jnp.maximum(jnp.minimum(cx - half, img_w - 1.0), 0.0)
            y1 = jnp.maximum(jnp.minimum(cy - half, img_h - 1.0), 0.0)
            x2 = jnp.maximum(jnp.minimum(cx + half, img_w), 1.0)
            y2 = jnp.maximum(jnp.minimum(cy + half, img_h), 1.0)
            c0v = iota16 * 0
            plsc.store_scatter(rois_t, [rows, c0v], bf, mask=ok)
            plsc.store_scatter(rois_t, [rows, c0v + 1], x1, mask=ok)
            plsc.store_scatter(rois_t, [rows, c0v + 2], y1, mask=ok)
            plsc.store_scatter(rois_t, [rows, c0v + 3], x2, mask=ok)
            plsc.store_scatter(rois_t, [rows, c0v + 4], y2, mask=ok)
            plsc.store_scatter(sco_t, [rows], vi, mask=ok)
            return 0
        lax.fori_loop(0, mc, _rank, 0)

        pltpu.sync_copy(rois_t.at[pl.ds(0, _K)],
                        rois_hbm.at[pl.ds(b * _K, _K)])
        pltpu.sync_copy(sco_t.at[pl.ds(0, _K)], sco_hbm.at[pl.ds(b * _K, _K)])


_sc_call = functools.partial(
    pl.kernel,
    out_type=(
        jax.ShapeDtypeStruct((_B * _K, 5), jnp.float32),
        jax.ShapeDtypeStruct((_B * _K,), jnp.float32),
    ),
    mesh=plsc.VectorSubcoreMesh(core_axis_name="c", subcore_axis_name="s"),
    compiler_params=pltpu.CompilerParams(needs_layout_passes=False,
                                        use_tc_tiling_on_sc=False),
    scratch_types=[
        pltpu.VMEM((_HW,), jnp.float32),
        pltpu.VMEM((_HW,), jnp.float32),
        pltpu.VMEM((_HW,), jnp.float32),
        pltpu.VMEM((_PAD_TILE,), jnp.float32),
        pltpu.VMEM((_CKSZ,), jnp.float32),
        pltpu.VMEM((_CKPAD + _CAP,), jnp.int32),
        pltpu.VMEM((_KPAD, 5), jnp.float32),
        pltpu.VMEM((_KPAD,), jnp.float32),
        pltpu.VMEM((16,), jnp.float32),
        pltpu.VMEM((16,), jnp.float32),
    ],
)(_body)


def kernel(route_logits, scale_logits, uncertainty_logits, image_h, image_w):
    B = route_logits.shape[0]
    r2 = route_logits.reshape(B, -1)
    s2 = scale_logits.reshape(B, -1)
    u2 = uncertainty_logits.reshape(B, -1)
    hvec = jnp.full((16,), jnp.asarray(image_h, jnp.float32))
    wvec = jnp.full((16,), jnp.asarray(image_w, jnp.float32))
    return _sc_call(r2, s2, u2, hvec, wvec)


# speedup vs baseline: 1.4610x; 1.0636x over previous
import functools

import jax
import jax.numpy as jnp
from jax import lax
from jax.experimental import pallas as pl
from jax.experimental.pallas import tpu as pltpu
from jax.experimental.pallas import tpu_sc as plsc

_STRIDE = 8.0
_MIN_SIZE = 16.0
_MAX_SIZE = 256.0

_B, _H, _W = 16, 64, 64
_HW = _H * _W
_PW = _W + 2
_PAD_TILE = ((_PW * (_H + 2) + 15) // 16) * 16
_CAP = 1040
_CKPAD = 16
_CKSZ = 1088
_K = 1000
_KPAD = 1008
_RT = 5040
_CHUNKS = _HW // 16


def _sigmoid(x):
    return 1.0 / (1.0 + jnp.exp(-x))


def _body(route_hbm, scale_hbm, unc_hbm, h_hbm, w_hbm, rois_hbm, sco_hbm,
          route_v, scale_v, unc_v, spad_v, ck_v, ci_v, rois_t, sco_t, h_v, w_v,
          pr_v, mrg_v, sh_pr):
    c = lax.axis_index("c")
    s = lax.axis_index("s")
    p = s & 1
    b = c * 8 + (s >> 1)

    if True:
        pltpu.sync_copy(route_hbm.at[b], route_v)
        pltpu.sync_copy(scale_hbm.at[b], scale_v)
        pltpu.sync_copy(unc_hbm.at[b], unc_v)
        pltpu.sync_copy(h_hbm, h_v)
        pltpu.sync_copy(w_hbm, w_v)

        iota16 = lax.iota(jnp.int32, 16)
        zf16 = jnp.zeros((16,), jnp.float32)

        def _fill(i, _):
            spad_v[pl.ds(i * 16, 16)] = zf16 - 1.0
            return 0
        lax.fori_loop(0, _PAD_TILE // 16, _fill, 0)

        def _score(t, _):
            r = route_v[pl.ds(t * 16, 16)]
            u = unc_v[pl.ds(t * 16, 16)]
            sr = _sigmoid(r)
            sc = sr * sr * (1.0 - 0.35 * _sigmoid(u))
            y = t >> 2
            q = t & 3
            spad_v[pl.ds((y + 1) * _PW + 1 + q * 16, 16)] = sc
            return 0
        lax.fori_loop(0, _CHUNKS, _score, 0)

        def _fillck(i, _):
            ck_v[pl.ds(i * 16, 16)] = zf16 - 1.0
            return 0
        lax.fori_loop(0, _CKSZ // 16, _fillck, 0)

        def _compact(t, off):
            y = t >> 2
            q = t & 3
            o = (y + 1) * _PW + 1 + q * 16
            sc = spad_v[pl.ds(o, 16)]
            m = jnp.maximum(spad_v[pl.ds(o - 1, 16)], spad_v[pl.ds(o + 1, 16)])
            m = jnp.maximum(m, jnp.maximum(spad_v[pl.ds(o - _PW - 1, 16)],
                                           spad_v[pl.ds(o - _PW, 16)]))
            m = jnp.maximum(m, jnp.maximum(spad_v[pl.ds(o - _PW + 1, 16)],
                                           spad_v[pl.ds(o + _PW - 1, 16)]))
            m = jnp.maximum(m, jnp.maximum(spad_v[pl.ds(o + _PW, 16)],
                                           spad_v[pl.ds(o + _PW + 1, 16)]))
            mask = sc >= m
            pos = off + plsc.cumsum(mask.astype(jnp.int32)) + (_CKPAD - 1)
            mask = mask & (pos < _CKPAD + _CAP)
            cnt = plsc.all_reduce_population_count(mask)
            idx = t * 16 + iota16
            plsc.store_scatter(ck_v, [pos], sc, mask=mask)
            plsc.store_scatter(ci_v, [pos], idx, mask=mask)
            return off + cnt
        offv = lax.fori_loop(0, _CHUNKS, _compact, jnp.zeros((16,), jnp.int32))
        m_count = jnp.max(offv)
        mc = (m_count + 15) >> 4

        def _zero(i, _):
            rows = i * 16 + iota16
            for c in range(5):
                plsc.store_scatter(rois_t, [rows, iota16 * 0 + c], zf16)
            return 0
        lax.fori_loop(0, _KPAD // 16, _zero, 0)

        def _zero2(i, _):
            sco_t[pl.ds(i * 16, 16)] = zf16
            return 0
        lax.fori_loop(0, _KPAD // 16, _zero2, 0)

        img_h = h_v[...]
        img_w = w_v[...]
        bf = zf16 + lax.convert_element_type(b, jnp.float32)

        def _prank(ci, _):
            di = _CKPAD + ci * 16
            vi = ck_v[pl.ds(di, 16)]

            def _ge(k, cnts):
                u = (2 * k + p) * 8
                acc = list(cnts)
                for j in range(8):
                    kj = ck_v[pl.ds(u + j, 16)]
                    acc[j & 3] = acc[j & 3] + (kj >= vi).astype(jnp.int32)
                return tuple(acc)

            def _gt(k, cnts):
                u = di + (2 * k + p) * 8
                acc = list(cnts)
                for j in range(8):
                    kj = ck_v[pl.ds(u + j, 16)]
                    acc[j & 3] = acc[j & 3] + (kj > vi).astype(jnp.int32)
                return tuple(acc)

            z4 = (jnp.zeros((16,), jnp.int32),) * 4
            nge = di >> 3
            ngt = (mc - ci) * 2
            cnts = lax.fori_loop(0, (nge - p + 1) >> 1, _ge, z4)
            c0, c1, c2, c3 = lax.fori_loop(0, (ngt - p + 1) >> 1, _gt, cnts)
            pr_v[pl.ds(ci * 16, 16)] = (c0 + c1) + (c2 + c3)
            return 0
        lax.fori_loop(0, mc, _prank, 0)

        @pl.when(p == 1)
        def _publish():
            pltpu.sync_copy(pr_v, sh_pr.at[s])
        plsc.subcore_barrier()

        @pl.when(p == 0)
        def _fetch():
            pltpu.sync_copy(sh_pr.at[s + 1], mrg_v)

        def _rank(ci, _):
            di = _CKPAD + ci * 16
            vi = ck_v[pl.ds(di, 16)]
            ii = ci_v[pl.ds(di, 16)]
            rank = pr_v[pl.ds(ci * 16, 16)] + mrg_v[pl.ds(ci * 16, 16)]

            ok = (vi > 0.0) & (rank < _K)
            rows = jnp.minimum(rank, _K)
            ii_safe = ii & (_HW - 1)
            xs = (ii_safe & (_W - 1)).astype(jnp.float32)
            ys = (ii_safe >> 6).astype(jnp.float32)
            cx = (xs + 0.5) * _STRIDE
            cy = (ys + 0.5) * _STRIDE
            sl = plsc.load_gather(scale_v, [ii_safe])
            ul = plsc.load_gather(unc_v, [ii_safe])
            side = (_MIN_SIZE + _sigmoid(sl) * (_MAX_SIZE - _MIN_SIZE)) \
                * (1.0 + 0.25 * _sigmoid(ul))
            half = side * 0.5
            x1 = jnp.maximum(jnp.minimum(cx - half, img_w - 1.0), 0.0)
            y1 = jnp.maximum(jnp.minimum(cy - half, img_h - 1.0), 0.0)
            x2 = jnp.maximum(jnp.minimum(cx + half, img_w), 1.0)
            y2 = jnp.maximum(jnp.minimum(cy + half, img_h), 1.0)
            c0v = iota16 * 0
            plsc.store_scatter(rois_t, [rows, c0v], bf, mask=ok)
            plsc.store_scatter(rois_t, [rows, c0v + 1], x1, mask=ok)
            plsc.store_scatter(rois_t, [rows, c0v + 2], y1, mask=ok)
            plsc.store_scatter(rois_t, [rows, c0v + 3], x2, mask=ok)
            plsc.store_scatter(rois_t, [rows, c0v + 4], y2, mask=ok)
            plsc.store_scatter(sco_t, [rows], vi, mask=ok)
            return 0

        @pl.when(p == 0)
        def _decode_out():
            lax.fori_loop(0, mc, _rank, 0)
            pltpu.sync_copy(rois_t.at[pl.ds(0, _K)],
                            rois_hbm.at[pl.ds(b * _K, _K)])
            pltpu.sync_copy(sco_t.at[pl.ds(0, _K)],
                            sco_hbm.at[pl.ds(b * _K, _K)])


_sc_call = functools.partial(
    pl.kernel,
    out_type=(
        jax.ShapeDtypeStruct((_B * _K, 5), jnp.float32),
        jax.ShapeDtypeStruct((_B * _K,), jnp.float32),
    ),
    mesh=plsc.VectorSubcoreMesh(core_axis_name="c", subcore_axis_name="s"),
    compiler_params=pltpu.CompilerParams(needs_layout_passes=False,
                                        use_tc_tiling_on_sc=False),
    scratch_types=[
        pltpu.VMEM((_HW,), jnp.float32),
        pltpu.VMEM((_HW,), jnp.float32),
        pltpu.VMEM((_HW,), jnp.float32),
        pltpu.VMEM((_PAD_TILE,), jnp.float32),
        pltpu.VMEM((_CKSZ,), jnp.float32),
        pltpu.VMEM((_CKPAD + _CAP,), jnp.int32),
        pltpu.VMEM((_KPAD, 5), jnp.float32),
        pltpu.VMEM((_KPAD,), jnp.float32),
        pltpu.VMEM((16,), jnp.float32),
        pltpu.VMEM((16,), jnp.float32),
        pltpu.VMEM((_CAP + _CKPAD,), jnp.int32),
        pltpu.VMEM((_CAP + _CKPAD,), jnp.int32),
        pltpu.VMEM_SHARED((16, _CAP + _CKPAD), jnp.int32),
    ],
)(_body)


def kernel(route_logits, scale_logits, uncertainty_logits, image_h, image_w):
    B = route_logits.shape[0]
    r2 = route_logits.reshape(B, -1)
    s2 = scale_logits.reshape(B, -1)
    u2 = uncertainty_logits.reshape(B, -1)
    hvec = jnp.full((16,), jnp.asarray(image_h, jnp.float32))
    wvec = jnp.full((16,), jnp.asarray(image_w, jnp.float32))
    return _sc_call(r2, s2, u2, hvec, wvec)
